# Initial kernel scaffold; baseline (speedup 1.0000x reference)
import jax
import jax.numpy as jnp
from jax.experimental import pallas as pl


def kernel(verts, faces):
    raise NotImplementedError("write your pallas kernel here")


# speedup vs baseline: 81.5312x; 81.5312x over previous
import functools

import jax
import jax.numpy as jnp
from jax import lax
from jax.experimental import pallas as pl
from jax.experimental.pallas import tpu as pltpu
from jax.experimental.pallas import tpu_sc as plsc

_L = 16
_NC = 2
_NS = 16
_NCH = 2


def _round_up(x, m):
    return (x + m - 1) // m * m


def _sc_accumulate(vtab, idx, T2, CH):
    TS = T2 // _NS
    FPT = _NCH * CH
    NPc = _NS * FPT
    C3 = 3 * CH
    mesh = plsc.VectorSubcoreMesh(core_axis_name="c", subcore_axis_name="s")

    @functools.partial(
        pl.kernel,
        out_type=jax.ShapeDtypeStruct((_NC * 4 * T2,), jnp.float32),
        mesh=mesh,
        scratch_types=[
            pltpu.VMEM((C3,), jnp.int32),
            pltpu.VMEM((C3,), jnp.float32),
            pltpu.VMEM((C3,), jnp.float32),
            pltpu.VMEM((C3,), jnp.float32),
            pltpu.VMEM_SHARED((T2,), jnp.float32),
            pltpu.VMEM_SHARED((T2,), jnp.float32),
            pltpu.VMEM_SHARED((T2,), jnp.float32),
            pltpu.VMEM_SHARED((T2,), jnp.float32),
            pltpu.VMEM_SHARED((T2,), jnp.float32),
            pltpu.VMEM_SHARED((T2,), jnp.float32),
            pltpu.VMEM_SHARED((T2,), jnp.float32),
        ],
    )
    def k(vtab_h, idx_h, p_h,
          idxall, gall, srep, ones,
          tvx, tvy, tvz, ax, ay, az, ac):
        c = lax.axis_index("c")
        s = lax.axis_index("s")

        zero16 = jnp.zeros((_L,), jnp.float32)
        one16 = jnp.ones((_L,), jnp.float32)

        def fill(i, _):
            sl = pl.ds(i * _L, _L)
            gall[sl] = zero16
            ones[sl] = one16
            return 0

        lax.fori_loop(0, C3 // _L, fill, 0)

        toff = s * TS
        tsl = pl.ds(toff, TS)
        zsl = pl.ds(0, TS)
        vbase = c * 3 * T2
        pltpu.sync_copy(vtab_h.at[pl.ds(vbase + toff, TS)], tvx.at[tsl])
        pltpu.sync_copy(vtab_h.at[pl.ds(vbase + T2 + toff, TS)], tvy.at[tsl])
        pltpu.sync_copy(vtab_h.at[pl.ds(vbase + 2 * T2 + toff, TS)], tvz.at[tsl])
        pltpu.sync_copy(gall.at[zsl], ax.at[tsl])
        pltpu.sync_copy(gall.at[zsl], ay.at[tsl])
        pltpu.sync_copy(gall.at[zsl], az.at[tsl])
        pltpu.sync_copy(gall.at[zsl], ac.at[tsl])
        plsc.subcore_barrier()

        ibase = c * 3 * NPc + s * FPT

        def chunk(ch, _):
            coff = ibase + ch * CH
            pltpu.sync_copy(idx_h.at[pl.ds(coff, CH)], idxall.at[pl.ds(0, CH)])
            pltpu.sync_copy(idx_h.at[pl.ds(coff + NPc, CH)],
                            idxall.at[pl.ds(CH, CH)])
            pltpu.sync_copy(idx_h.at[pl.ds(coff + 2 * NPc, CH)],
                            idxall.at[pl.ds(2 * CH, CH)])

            for tv, acc in ((tvx, ax), (tvy, ay), (tvz, az)):
                pltpu.sync_copy(tv.at[idxall], gall)

                def addsum(i, _):
                    sl0 = pl.ds(i * _L, _L)
                    sl1 = pl.ds(CH + i * _L, _L)
                    sl2 = pl.ds(2 * CH + i * _L, _L)
                    s16 = gall[sl0] + gall[sl1] + gall[sl2]
                    srep[sl0] = s16
                    srep[sl1] = s16
                    srep[sl2] = s16
                    return 0

                lax.fori_loop(0, CH // _L, addsum, 0)
                pltpu.sync_copy(srep, acc.at[idxall], add=True)

            pltpu.sync_copy(ones, ac.at[idxall], add=True)
            return 0

        lax.fori_loop(0, _NCH, chunk, 0)

        plsc.subcore_barrier()

        pbase = c * 4 * T2
        pltpu.sync_copy(ax.at[tsl], p_h.at[pl.ds(pbase + toff, TS)])
        pltpu.sync_copy(ay.at[tsl], p_h.at[pl.ds(pbase + T2 + toff, TS)])
        pltpu.sync_copy(az.at[tsl], p_h.at[pl.ds(pbase + 2 * T2 + toff, TS)])
        pltpu.sync_copy(ac.at[tsl], p_h.at[pl.ds(pbase + 3 * T2 + toff, TS)])

    return k(vtab, idx)


def _combine(p8, v6, T2):
    BLK = 2048

    def body(p_ref, v_ref, o_ref):
        p = p_ref[...]
        v = v_ref[...]
        cnt0 = p[3:4]
        cnt1 = p[7:8]
        o_ref[0:3, :] = (3.0 * cnt0 * v[0:3] - p[0:3]) / (2.0 * cnt0 + 1e-12)
        o_ref[3:6, :] = (3.0 * cnt1 * v[3:6] - p[4:7]) / (2.0 * cnt1 + 1e-12)

    return pl.pallas_call(
        body,
        grid=(T2 // BLK,),
        in_specs=[
            pl.BlockSpec((2 * 4, BLK), lambda i: (0, i)),
            pl.BlockSpec((6, BLK), lambda i: (0, i)),
        ],
        out_specs=pl.BlockSpec((6, BLK), lambda i: (0, i)),
        out_shape=jax.ShapeDtypeStruct((6, T2), jnp.float32),
    )(p8, v6)


def kernel(verts, faces):
    b, nv, d = verts.shape
    nf = faces.shape[1]
    n2 = b * nv // _NC
    spc = b * nf // _NC

    T2 = _round_up(n2 + 1, 2048)
    FPT = _round_up(-(-spc // _NS), _NCH * 128)
    CH = FPT // _NCH
    NPc = _NS * FPT

    v2 = verts.reshape(_NC, n2, d)
    vtab = jnp.pad(v2, ((0, 0), (0, T2 - n2), (0, 0))).transpose(0, 2, 1)

    local_off = ((jnp.arange(b, dtype=faces.dtype) % (b // _NC)) * nv)
    f = (faces + local_off.reshape(-1, 1, 1)).reshape(_NC, spc, 3)
    f = jnp.pad(f, ((0, 0), (0, NPc - spc), (0, 0)), constant_values=n2)
    idx = f.transpose(0, 2, 1)

    p = _sc_accumulate(vtab.reshape(-1), idx.reshape(-1), T2, CH)
    xs = _combine(p.reshape(_NC * 4, T2), vtab.reshape(_NC * 3, T2), T2)
    return xs.reshape(_NC, 3, T2)[:, :, :n2].transpose(0, 2, 1).reshape(b, nv, d)


# speedup vs baseline: 108.5628x; 1.3315x over previous
import functools

import jax
import jax.numpy as jnp
from jax import lax
from jax.experimental import pallas as pl
from jax.experimental.pallas import tpu as pltpu
from jax.experimental.pallas import tpu_sc as plsc

_L = 16
_NC = 2
_NS = 16
_NCH = 2


def _round_up(x, m):
    return (x + m - 1) // m * m


def _sc_accumulate(vtab, idx, T2, CH):
    TS = T2 // _NS
    FPT = _NCH * CH
    NPc = _NS * FPT
    C3 = 3 * CH
    mesh = plsc.VectorSubcoreMesh(core_axis_name="c", subcore_axis_name="s")

    @functools.partial(
        pl.kernel,
        out_type=jax.ShapeDtypeStruct((_NC * 4 * T2,), jnp.float32),
        mesh=mesh,
        scratch_types=[
            pltpu.VMEM((C3,), jnp.int32),
            pltpu.VMEM((C3,), jnp.float32),
            pltpu.VMEM((C3,), jnp.float32),
            pltpu.VMEM_SHARED((T2,), jnp.float32),
            pltpu.VMEM_SHARED((T2,), jnp.float32),
            pltpu.VMEM_SHARED((T2,), jnp.float32),
            pltpu.VMEM_SHARED((T2,), jnp.float32),
            pltpu.VMEM_SHARED((T2,), jnp.float32),
            pltpu.VMEM_SHARED((T2,), jnp.float32),
            pltpu.VMEM_SHARED((T2,), jnp.float32),
            pltpu.SemaphoreType.DMA,
            pltpu.SemaphoreType.DMA,
        ],
    )
    def k(vtab_h, idx_h, p_h,
          idxall, g, sr,
          tvx, tvy, tvz, ax, ay, az, ac,
          sem_g, sem_s):
        c = lax.axis_index("c")
        s = lax.axis_index("s")

        zero16 = jnp.zeros((_L,), jnp.float32)

        def fillz(i, _):
            sr[pl.ds(i * _L, _L)] = zero16
            return 0

        lax.fori_loop(0, TS // _L, fillz, 0)

        toff = s * TS
        tsl = pl.ds(toff, TS)
        zsl = pl.ds(0, TS)
        vbase = c * 3 * T2
        pltpu.sync_copy(vtab_h.at[pl.ds(vbase + toff, TS)], tvx.at[tsl])
        pltpu.sync_copy(vtab_h.at[pl.ds(vbase + T2 + toff, TS)], tvy.at[tsl])
        pltpu.sync_copy(vtab_h.at[pl.ds(vbase + 2 * T2 + toff, TS)], tvz.at[tsl])
        pltpu.sync_copy(sr.at[zsl], ax.at[tsl])
        pltpu.sync_copy(sr.at[zsl], ay.at[tsl])
        pltpu.sync_copy(sr.at[zsl], az.at[tsl])
        pltpu.sync_copy(sr.at[zsl], ac.at[tsl])
        plsc.subcore_barrier()

        ibase = c * 3 * NPc + s * FPT
        one16 = jnp.ones((_L,), jnp.float32)

        def chunk(ch, _):
            coff = ibase + ch * CH
            pltpu.sync_copy(idx_h.at[pl.ds(coff, CH)], idxall.at[pl.ds(0, CH)])
            pltpu.sync_copy(idx_h.at[pl.ds(coff + NPc, CH)],
                            idxall.at[pl.ds(CH, CH)])
            pltpu.sync_copy(idx_h.at[pl.ds(coff + 2 * NPc, CH)],
                            idxall.at[pl.ds(2 * CH, CH)])

            def addsum(i, _):
                sl0 = pl.ds(i * _L, _L)
                sl1 = pl.ds(CH + i * _L, _L)
                sl2 = pl.ds(2 * CH + i * _L, _L)
                s16 = g[sl0] + g[sl1] + g[sl2]
                sr[sl0] = s16
                sr[sl1] = s16
                sr[sl2] = s16
                return 0

            pltpu.async_copy(tvx.at[idxall], g, sem_g).wait()
            lax.fori_loop(0, CH // _L, addsum, 0)
            sc_x = pltpu.async_copy(sr, ax.at[idxall], sem_s, add=True)

            g_y = pltpu.async_copy(tvy.at[idxall], g, sem_g)
            g_y.wait()
            sc_x.wait()
            lax.fori_loop(0, CH // _L, addsum, 0)
            sc_y = pltpu.async_copy(sr, ay.at[idxall], sem_s, add=True)

            g_z = pltpu.async_copy(tvz.at[idxall], g, sem_g)
            g_z.wait()
            sc_y.wait()
            lax.fori_loop(0, CH // _L, addsum, 0)
            sc_z = pltpu.async_copy(sr, az.at[idxall], sem_s, add=True)

            def fillo(i, _):
                g[pl.ds(i * _L, _L)] = one16
                return 0

            lax.fori_loop(0, C3 // _L, fillo, 0)
            sc_c = pltpu.async_copy(g, ac.at[idxall], sem_s, add=True)
            sc_z.wait()
            sc_c.wait()
            return 0

        lax.fori_loop(0, _NCH, chunk, 0)
        plsc.subcore_barrier()

        pbase = c * 4 * T2
        pltpu.sync_copy(ax.at[tsl], p_h.at[pl.ds(pbase + toff, TS)])
        pltpu.sync_copy(ay.at[tsl], p_h.at[pl.ds(pbase + T2 + toff, TS)])
        pltpu.sync_copy(az.at[tsl], p_h.at[pl.ds(pbase + 2 * T2 + toff, TS)])
        pltpu.sync_copy(ac.at[tsl], p_h.at[pl.ds(pbase + 3 * T2 + toff, TS)])

    return k(vtab, idx)


def _combine(p8, v6, T2):
    BLK = 14336

    def body(p_ref, v_ref, o_ref):
        p = p_ref[...]
        v = v_ref[...]
        cnt0 = p[3:4]
        cnt1 = p[7:8]
        o_ref[0:3, :] = (3.0 * cnt0 * v[0:3] - p[0:3]) / (2.0 * cnt0 + 1e-12)
        o_ref[3:6, :] = (3.0 * cnt1 * v[3:6] - p[4:7]) / (2.0 * cnt1 + 1e-12)

    return pl.pallas_call(
        body,
        grid=(T2 // BLK,),
        in_specs=[
            pl.BlockSpec((2 * 4, BLK), lambda i: (0, i)),
            pl.BlockSpec((6, BLK), lambda i: (0, i)),
        ],
        out_specs=pl.BlockSpec((6, BLK), lambda i: (0, i)),
        out_shape=jax.ShapeDtypeStruct((6, T2), jnp.float32),
    )(p8, v6)


def kernel(verts, faces):
    b, nv, d = verts.shape
    nf = faces.shape[1]
    n2 = b * nv // _NC
    spc = b * nf // _NC

    T2 = _round_up(n2 + 1, 2048)
    CH = _round_up(-(-spc // _NS), _NCH * _L) // _NCH

    v2 = verts.reshape(_NC, n2, d)
    vtab = jnp.pad(v2, ((0, 0), (0, T2 - n2), (0, 0))).transpose(0, 2, 1)

    NPc = _NS * _NCH * CH
    local_off = ((jnp.arange(b, dtype=faces.dtype) % (b // _NC)) * nv)
    f = (faces + local_off.reshape(-1, 1, 1)).reshape(_NC, spc, 3)
    f = jnp.pad(f, ((0, 0), (0, NPc - spc), (0, 0)), constant_values=n2)
    idx = f.transpose(0, 2, 1)

    p = _sc_accumulate(vtab.reshape(-1), idx.reshape(-1), T2, CH)
    xs = _combine(p.reshape(_NC * 4, T2), vtab.reshape(_NC * 3, T2), T2)
    return xs.reshape(_NC, 3, T2)[:, :, :n2].transpose(0, 2, 1).reshape(b, nv, d)
